# baseline (speedup 1.0000x reference)
import functools

import jax
import jax.numpy as jnp
from jax import lax
from jax.experimental import pallas as pl
from jax.experimental.pallas import tpu as pltpu
from jax.experimental.pallas import tpu_sc as plsc

_NUM_WORKERS = 32
_VCHUNK = 64
_NCHUNKS = 4
_BLK = 2048


def _sc_gather(value_table, vids, Bc, H, row0):
    rows_per_w = Bc // _NUM_WORKERS
    nvc = rows_per_w // _VCHUNK
    mesh = plsc.VectorSubcoreMesh(core_axis_name="c", subcore_axis_name="s")

    @functools.partial(
        pl.kernel,
        mesh=mesh,
        out_type=jax.ShapeDtypeStruct((Bc, H), jnp.float32),
        scratch_types=[
            pltpu.VMEM((rows_per_w,), jnp.int32),
            pltpu.VMEM((_VCHUNK, H), jnp.float32),
            pltpu.VMEM((_VCHUNK, H), jnp.float32),
            pltpu.SemaphoreType.DMA,
            pltpu.SemaphoreType.DMA,
        ],
    )
    def k(table_hbm, vids_hbm, emb_hbm, vidx, rows0, rows1, sem0, sem1):
        wid = lax.axis_index("s") * 2 + lax.axis_index("c")
        pltpu.sync_copy(vids_hbm.at[pl.ds(row0 + wid * rows_per_w, rows_per_w)],
                        vidx)
        base = wid * rows_per_w
        bufs = (rows0, rows1)
        sems = (sem0, sem1)
        copies = [
            pltpu.make_async_copy(
                table_hbm.at[vidx.at[pl.ds(c * _VCHUNK, _VCHUNK)]],
                bufs[c % 2], sems[c % 2])
            for c in range(nvc)
        ]
        copies[0].start()
        for c in range(nvc):
            copies[c].wait()
            if c + 1 < nvc:
                copies[c + 1].start()
            pltpu.sync_copy(bufs[c % 2], emb_hbm.at[pl.ds(base + c * _VCHUNK, _VCHUNK)])

    return k(value_table, vids)


def _tc_dense(emb_c, W, b, g, be, eids_c, etab, node_acc, eattr_acc, row0, Btot):
    Bc, H = emb_c.shape
    n_types, EA = etab.shape
    nsteps = Bc // _BLK
    base = row0 // _BLK
    aliased = node_acc is not None

    def body(*refs):
        if aliased:
            refs = refs[2:]
        (emb_ref, w_ref, b_ref, g_ref, be_ref, eid_ref, etab_t_ref,
         out_ref, eattr_ref) = refs
        h = jnp.dot(emb_ref[...], w_ref[...], preferred_element_type=jnp.float32)
        h = h + b_ref[...]
        mean = jnp.mean(h, axis=-1, keepdims=True)
        var = jnp.mean((h - mean) ** 2, axis=-1, keepdims=True)
        out_ref[...] = (h - mean) * lax.rsqrt(var + 1e-5) * g_ref[...] + be_ref[...]
        ids = eid_ref[0, 0, :][None, :]
        onehot = (ids == lax.broadcasted_iota(jnp.int32, (n_types, _BLK), 0))
        eattr_ref[...] = jnp.dot(etab_t_ref[...], onehot.astype(jnp.float32),
                                 preferred_element_type=jnp.float32)

    in_specs = [
        pl.BlockSpec((_BLK, H), lambda i: (i, 0)),
        pl.BlockSpec((H, H), lambda i: (0, 0)),
        pl.BlockSpec((1, H), lambda i: (0, 0)),
        pl.BlockSpec((1, H), lambda i: (0, 0)),
        pl.BlockSpec((1, H), lambda i: (0, 0)),
        pl.BlockSpec((1, 1, _BLK), lambda i: (base + i, 0, 0)),
        pl.BlockSpec((EA, n_types), lambda i: (0, 0)),
    ]
    args = [emb_c, W, b.reshape(1, H), g.reshape(1, H), be.reshape(1, H),
            eids_c, etab.T]
    aliases = {}
    if aliased:
        in_specs = [pl.BlockSpec(memory_space=pl.ANY),
                    pl.BlockSpec(memory_space=pl.ANY)] + in_specs
        args = [node_acc, eattr_acc] + args
        aliases = {0: 0, 1: 1}
    return pl.pallas_call(
        body,
        grid=(nsteps,),
        in_specs=in_specs,
        out_specs=[
            pl.BlockSpec((_BLK, H), lambda i: (base + i, 0)),
            pl.BlockSpec((EA, _BLK), lambda i: (0, base + i)),
        ],
        out_shape=[
            jax.ShapeDtypeStruct((Btot, H), jnp.float32),
            jax.ShapeDtypeStruct((EA, Btot), jnp.float32),
        ],
        input_output_aliases=aliases,
    )(*args)


def kernel(value_ids, edge_type_ids, value_table, W_proj, b_proj, ln_gamma,
           ln_beta, edge_type_table):
    B = value_ids.shape[0]
    E = edge_type_ids.shape[0]
    H = value_table.shape[1]
    vids = value_ids.astype(jnp.int32)
    eids = edge_type_ids.astype(jnp.int32)

    eids3 = eids.reshape(E // _BLK, 1, _BLK)

    sizes = [B // 8, B // 4, B // 4, 3 * B // 8]
    offs = [0]
    for s in sizes[:-1]:
        offs.append(offs[-1] + s)

    embs = [_sc_gather(value_table, vids, s, H, o)
            for s, o in zip(sizes, offs)]

    node_acc, eattr_acc = None, None
    for emb_c, o in zip(embs, offs):
        node_acc, eattr_acc = _tc_dense(
            emb_c, W_proj, b_proj, ln_gamma, ln_beta, eids3,
            edge_type_table, node_acc, eattr_acc, o, B)
    return node_acc, eattr_acc.T


# speedup vs baseline: 1.0296x; 1.0296x over previous
import functools

import jax
import jax.numpy as jnp
from jax import lax
from jax.experimental import pallas as pl
from jax.experimental.pallas import tpu as pltpu
from jax.experimental.pallas import tpu_sc as plsc

_NUM_WORKERS = 32
_VCHUNK = 64
_NCHUNKS = 4
_BLK = 2048


def _sc_gather(value_table, vids, Bc, H, row0):
    rows_per_w = Bc // _NUM_WORKERS
    nvc = rows_per_w // _VCHUNK
    mesh = plsc.VectorSubcoreMesh(core_axis_name="c", subcore_axis_name="s")

    @functools.partial(
        pl.kernel,
        mesh=mesh,
        out_type=jax.ShapeDtypeStruct((Bc, H), jnp.float32),
        scratch_types=[
            pltpu.VMEM((rows_per_w,), jnp.int32),
            pltpu.VMEM((_VCHUNK, H), jnp.float32),
            pltpu.VMEM((_VCHUNK, H), jnp.float32),
            pltpu.SemaphoreType.DMA,
            pltpu.SemaphoreType.DMA,
        ],
    )
    def k(table_hbm, vids_hbm, emb_hbm, vidx, rows0, rows1, sem0, sem1):
        wid = lax.axis_index("s") * 2 + lax.axis_index("c")
        pltpu.sync_copy(vids_hbm.at[pl.ds(row0 + wid * rows_per_w, rows_per_w)],
                        vidx)
        base = wid * rows_per_w
        bufs = (rows0, rows1)
        sems = (sem0, sem1)
        copies = [
            pltpu.make_async_copy(
                table_hbm.at[vidx.at[pl.ds(c * _VCHUNK, _VCHUNK)]],
                bufs[c % 2], sems[c % 2])
            for c in range(nvc)
        ]
        copies[0].start()
        for c in range(nvc):
            copies[c].wait()
            if c + 1 < nvc:
                copies[c + 1].start()
            pltpu.sync_copy(bufs[c % 2], emb_hbm.at[pl.ds(base + c * _VCHUNK, _VCHUNK)])

    return k(value_table, vids)


def _tc_dense(emb_c, W, b, g, be, eids_c, etab, node_acc, eattr_acc, row0, Btot):
    Bc, H = emb_c.shape
    n_types, EA = etab.shape
    nsteps = Bc // _BLK
    base = row0 // _BLK
    aliased = node_acc is not None

    def body(*refs):
        if aliased:
            refs = refs[2:]
        (emb_ref, w_ref, b_ref, g_ref, be_ref, eid_ref, etab_t_ref,
         out_ref, eattr_ref) = refs
        h = jnp.dot(emb_ref[...].astype(jnp.bfloat16), w_ref[...],
                    preferred_element_type=jnp.float32)
        h = h + b_ref[...]
        mean = jnp.mean(h, axis=-1, keepdims=True)
        var = jnp.mean((h - mean) ** 2, axis=-1, keepdims=True)
        out_ref[...] = (h - mean) * lax.rsqrt(var + 1e-5) * g_ref[...] + be_ref[...]
        ids = eid_ref[0, 0, :][None, :]
        onehot = (ids == lax.broadcasted_iota(jnp.int32, (n_types, _BLK), 0))
        eattr_ref[...] = jnp.dot(etab_t_ref[...], onehot.astype(jnp.float32),
                                 preferred_element_type=jnp.float32)

    in_specs = [
        pl.BlockSpec((_BLK, H), lambda i: (i, 0)),
        pl.BlockSpec((H, H), lambda i: (0, 0)),
        pl.BlockSpec((1, H), lambda i: (0, 0)),
        pl.BlockSpec((1, H), lambda i: (0, 0)),
        pl.BlockSpec((1, H), lambda i: (0, 0)),
        pl.BlockSpec((1, 1, _BLK), lambda i: (base + i, 0, 0)),
        pl.BlockSpec((EA, n_types), lambda i: (0, 0)),
    ]
    args = [emb_c, W.astype(jnp.bfloat16), b.reshape(1, H), g.reshape(1, H),
            be.reshape(1, H), eids_c, etab.T]
    aliases = {}
    if aliased:
        in_specs = [pl.BlockSpec(memory_space=pl.ANY),
                    pl.BlockSpec(memory_space=pl.ANY)] + in_specs
        args = [node_acc, eattr_acc] + args
        aliases = {0: 0, 1: 1}
    return pl.pallas_call(
        body,
        grid=(nsteps,),
        in_specs=in_specs,
        out_specs=[
            pl.BlockSpec((_BLK, H), lambda i: (base + i, 0)),
            pl.BlockSpec((EA, _BLK), lambda i: (0, base + i)),
        ],
        out_shape=[
            jax.ShapeDtypeStruct((Btot, H), jnp.float32),
            jax.ShapeDtypeStruct((EA, Btot), jnp.float32),
        ],
        input_output_aliases=aliases,
    )(*args)


def kernel(value_ids, edge_type_ids, value_table, W_proj, b_proj, ln_gamma,
           ln_beta, edge_type_table):
    B = value_ids.shape[0]
    E = edge_type_ids.shape[0]
    H = value_table.shape[1]
    vids = value_ids.astype(jnp.int32)
    eids = edge_type_ids.astype(jnp.int32)

    eids3 = eids.reshape(E // _BLK, 1, _BLK)

    sizes = [B // 4] * _NCHUNKS
    offs = [0]
    for s in sizes[:-1]:
        offs.append(offs[-1] + s)

    embs = [_sc_gather(value_table, vids, s, H, o)
            for s, o in zip(sizes, offs)]

    node_acc, eattr_acc = None, None
    for emb_c, o in zip(embs, offs):
        node_acc, eattr_acc = _tc_dense(
            emb_c, W_proj, b_proj, ln_gamma, ln_beta, eids3,
            edge_type_table, node_acc, eattr_acc, o, B)
    return node_acc, eattr_acc.T


# speedup vs baseline: 1.0348x; 1.0050x over previous
import functools

import jax
import jax.numpy as jnp
from jax import lax
from jax.experimental import pallas as pl
from jax.experimental.pallas import tpu as pltpu
from jax.experimental.pallas import tpu_sc as plsc

_NUM_WORKERS = 32
_VCHUNK = 64
_NCHUNKS = 4
_BLK = 2048


def _sc_gather(value_table, vids, Bc, H, row0):
    rows_per_w = Bc // _NUM_WORKERS
    nvc = rows_per_w // _VCHUNK
    mesh = plsc.VectorSubcoreMesh(core_axis_name="c", subcore_axis_name="s")

    @functools.partial(
        pl.kernel,
        mesh=mesh,
        out_type=jax.ShapeDtypeStruct((Bc, H), jnp.float32),
        scratch_types=[
            pltpu.VMEM((rows_per_w,), jnp.int32),
            pltpu.VMEM((_VCHUNK, H), jnp.float32),
            pltpu.VMEM((_VCHUNK, H), jnp.float32),
            pltpu.SemaphoreType.DMA,
            pltpu.SemaphoreType.DMA,
        ],
    )
    def k(table_hbm, vids_hbm, emb_hbm, vidx, rows0, rows1, sem0, sem1):
        wid = lax.axis_index("s") * 2 + lax.axis_index("c")
        pltpu.sync_copy(vids_hbm.at[pl.ds(row0 + wid * rows_per_w, rows_per_w)],
                        vidx)
        base = wid * rows_per_w
        bufs = (rows0, rows1)
        sems = (sem0, sem1)
        copies = [
            pltpu.make_async_copy(
                table_hbm.at[vidx.at[pl.ds(c * _VCHUNK, _VCHUNK)]],
                bufs[c % 2], sems[c % 2])
            for c in range(nvc)
        ]
        copies[0].start()
        for c in range(nvc):
            copies[c].wait()
            if c + 1 < nvc:
                copies[c + 1].start()
            pltpu.sync_copy(bufs[c % 2], emb_hbm.at[pl.ds(base + c * _VCHUNK, _VCHUNK)])

    return k(value_table, vids)


def _tc_dense(emb_c, W, b, g, be, eids_c, etab, node_acc, eattr_acc, row0, Btot):
    Bc, H = emb_c.shape
    n_types, EA = etab.shape
    nsteps = Bc // _BLK
    base = row0 // _BLK
    aliased = node_acc is not None

    def body(*refs):
        if aliased:
            refs = refs[2:]
        (emb_ref, w_ref, b_ref, g_ref, be_ref, eid_ref, etab_t_ref,
         out_ref, eattr_ref) = refs
        h = jnp.dot(emb_ref[...].astype(jnp.bfloat16), w_ref[...],
                    preferred_element_type=jnp.float32)
        h = h + b_ref[...]
        mean = jnp.mean(h, axis=-1, keepdims=True)
        var = jnp.mean((h - mean) ** 2, axis=-1, keepdims=True)
        out_ref[...] = (h - mean) * lax.rsqrt(var + 1e-5) * g_ref[...] + be_ref[...]
        ids = eid_ref[0, 0, :][None, :]
        onehot = (ids == lax.broadcasted_iota(jnp.int32, (n_types, _BLK), 0))
        eattr_ref[...] = jnp.dot(etab_t_ref[...], onehot.astype(jnp.float32),
                                 preferred_element_type=jnp.float32)

    in_specs = [
        pl.BlockSpec((_BLK, H), lambda i: (i, 0)),
        pl.BlockSpec((H, H), lambda i: (0, 0)),
        pl.BlockSpec((1, H), lambda i: (0, 0)),
        pl.BlockSpec((1, H), lambda i: (0, 0)),
        pl.BlockSpec((1, H), lambda i: (0, 0)),
        pl.BlockSpec((1, 1, _BLK), lambda i: (base + i, 0, 0)),
        pl.BlockSpec((EA, n_types), lambda i: (0, 0)),
    ]
    args = [emb_c, W.astype(jnp.bfloat16), b.reshape(1, H), g.reshape(1, H),
            be.reshape(1, H), eids_c, etab.T]
    aliases = {}
    if aliased:
        in_specs = [pl.BlockSpec(memory_space=pl.ANY),
                    pl.BlockSpec(memory_space=pl.ANY)] + in_specs
        args = [node_acc, eattr_acc] + args
        aliases = {0: 0, 1: 1}
    return pl.pallas_call(
        body,
        grid=(nsteps,),
        in_specs=in_specs,
        out_specs=[
            pl.BlockSpec((_BLK, H), lambda i: (base + i, 0)),
            pl.BlockSpec((EA, _BLK), lambda i: (0, base + i)),
        ],
        out_shape=[
            jax.ShapeDtypeStruct((Btot, H), jnp.float32),
            jax.ShapeDtypeStruct((EA, Btot), jnp.float32),
        ],
        input_output_aliases=aliases,
    )(*args)


def kernel(value_ids, edge_type_ids, value_table, W_proj, b_proj, ln_gamma,
           ln_beta, edge_type_table):
    B = value_ids.shape[0]
    E = edge_type_ids.shape[0]
    H = value_table.shape[1]
    vids = value_ids.astype(jnp.int32)
    eids = edge_type_ids.astype(jnp.int32)

    eids3 = eids.reshape(E // _BLK, 1, _BLK)

    sizes = [B // _NCHUNKS] * _NCHUNKS
    offs = [0]
    for s in sizes[:-1]:
        offs.append(offs[-1] + s)

    embs = [_sc_gather(value_table, vids, s, H, o)
            for s, o in zip(sizes, offs)]

    node_acc, eattr_acc = None, None
    for emb_c, o in zip(embs, offs):
        node_acc, eattr_acc = _tc_dense(
            emb_c, W_proj, b_proj, ln_gamma, ln_beta, eids3,
            edge_type_table, node_acc, eattr_acc, o, B)
    return node_acc, eattr_acc.T
